# baseline (speedup 1.0000x reference)
import jax
import jax.numpy as jnp
from jax.experimental import pallas as pl
from jax.experimental.pallas import tpu as pltpu
from jax.experimental.pallas import tpu_sc as plsc

D = 256
K = 8192
ROW_BLK = 512
GATHER_WINDOW = 128


ARG_CHUNK = 2048


def argmin_block_kernel(x2_ref, w2_ref, x_ref, w_ref, idx_ref):
    x_blk = x_ref[...]
    w = w_ref[...]
    mm2 = jax.lax.dot_general(
        x_blk + x_blk, w, (((1,), (1,)), ((), ())),
        preferred_element_type=jnp.float32)
    scores = (x2_ref[...] + w2_ref[...]) - mm2
    big = jnp.int32(2147483647)
    acc_v = None
    for c in range(K // ARG_CHUNK):
        s = scores[:, c * ARG_CHUNK:(c + 1) * ARG_CHUNK]
        m = jnp.min(s, axis=1, keepdims=True)
        lane = jax.lax.broadcasted_iota(jnp.int32, s.shape, 1) + c * ARG_CHUNK
        i = jnp.min(jnp.where(s == m, lane, big), axis=1, keepdims=True)
        if acc_v is None:
            acc_v = m.astype(jnp.bfloat16).astype(jnp.float32)
            acc_i = i
        else:
            keep = acc_v < m
            tie = acc_v == m
            acc_v = jnp.where(keep, acc_v, m).astype(
                jnp.bfloat16).astype(jnp.float32)
            acc_i = jnp.where(keep | tie, acc_i, i)
    idx_ref[...] = acc_i


def residual_kernel(q_ref, x_ref, o_ref):
    o_ref[...] = (q_ref[...] - x_ref[...]) + x_ref[...]


def sc_gather_rows(table, idx_row):
    n = idx_row.shape[1]
    mesh = plsc.VectorSubcoreMesh(
        core_axis_name="core", subcore_axis_name="subcore")

    @pl.kernel(out_type=jax.ShapeDtypeStruct((n, D), table.dtype), mesh=mesh)
    def gather_kernel(w_hbm, i_hbm, o_hbm):
        def body(i_vmem, o_vmem):
            pltpu.sync_copy(w_hbm.at[i_vmem.at[0]], o_vmem)

        pltpu.emit_pipeline(
            body,
            grid=(n // GATHER_WINDOW,),
            in_specs=[pl.BlockSpec((1, GATHER_WINDOW), lambda i: (0, i))],
            out_specs=[pl.BlockSpec((GATHER_WINDOW, D), lambda i: (i, 0))],
            core_axis_name="subcore",
            dimension_semantics=(pltpu.PARALLEL,),
        )(i_hbm, o_hbm)

    return gather_kernel(table, idx_row)


def _argmin_call(x2_h, w2row, x_h, W):
    nh = x_h.shape[0]
    return pl.pallas_call(
        argmin_block_kernel,
        grid=(nh // ROW_BLK,),
        in_specs=[
            pl.BlockSpec((ROW_BLK, 1), lambda i: (i, 0)),
            pl.BlockSpec((1, K), lambda i: (0, 0)),
            pl.BlockSpec((ROW_BLK, D), lambda i: (i, 0)),
            pl.BlockSpec((K, D), lambda i: (0, 0)),
        ],
        out_specs=pl.BlockSpec((ROW_BLK, 1), lambda i: (i, 0)),
        out_shape=jax.ShapeDtypeStruct((nh, 1), jnp.int32),
    )(x2_h, w2row, x_h, W)


def _residual_call(q_h, x_h):
    nh = x_h.shape[0]
    return pl.pallas_call(
        residual_kernel,
        grid=(4,),
        in_specs=[
            pl.BlockSpec((nh // 4, D), lambda i: (i, 0)),
            pl.BlockSpec((nh // 4, D), lambda i: (i, 0)),
        ],
        out_specs=pl.BlockSpec((nh // 4, D), lambda i: (i, 0)),
        out_shape=jax.ShapeDtypeStruct((nh, D), jnp.float32),
    )(q_h, x_h)


HALVES = 2


def kernel(x, W):
    flat_x = x.reshape(-1, x.shape[-1])
    n = flat_x.shape[0]
    nh = n // HALVES
    x2 = jnp.sum(flat_x ** 2, axis=1, keepdims=True)
    w2 = jnp.sum(W ** 2, axis=1)
    w2row = w2.reshape(1, K)

    qs, outs = [], []
    for h in range(HALVES):
        sl = slice(h * nh, (h + 1) * nh)
        idx_h = _argmin_call(x2[sl], w2row, flat_x[sl], W)
        q_h = sc_gather_rows(W, idx_h.reshape(1, nh))
        qs.append(q_h)
        outs.append(_residual_call(q_h, flat_x[sl]))

    q = jnp.concatenate(qs, axis=0)
    out2_flat = jnp.concatenate(outs, axis=0)
    return (q.reshape(x.shape), out2_flat.reshape(x.shape))


# speedup vs baseline: 1.1980x; 1.1980x over previous
import jax
import jax.numpy as jnp
from jax.experimental import pallas as pl
from jax.experimental.pallas import tpu as pltpu
from jax.experimental.pallas import tpu_sc as plsc

D = 256
K = 8192
ROW_BLK = 512
GATHER_WINDOW = 128


ARG_CHUNK = 2048


def argmin_block_kernel(x2_ref, w2_ref, x_ref, w_ref, idx_ref):
    x_blk = x_ref[...]
    w = w_ref[...]
    mm2 = jax.lax.dot_general(
        x_blk + x_blk, w, (((1,), (1,)), ((), ())),
        preferred_element_type=jnp.float32)
    scores = (x2_ref[...] + w2_ref[...]) - mm2
    big = jnp.int32(2147483647)
    acc_v = None
    for c in range(K // ARG_CHUNK):
        s = scores[:, c * ARG_CHUNK:(c + 1) * ARG_CHUNK]
        m = jnp.min(s, axis=1, keepdims=True)
        i = jnp.argmin(s, axis=1, keepdims=True).astype(jnp.int32) + c * ARG_CHUNK
        if acc_v is None:
            acc_v = m.astype(jnp.bfloat16).astype(jnp.float32)
            acc_i = i
        else:
            keep = acc_v < m
            tie = acc_v == m
            acc_v = jnp.where(keep, acc_v, m).astype(
                jnp.bfloat16).astype(jnp.float32)
            acc_i = jnp.where(keep | tie, acc_i, i)
    idx_ref[...] = acc_i


def residual_kernel(q_ref, x_ref, o_ref):
    o_ref[...] = (q_ref[...] - x_ref[...]) + x_ref[...]


def sc_gather_rows(table, idx_row):
    n = idx_row.shape[1]
    mesh = plsc.VectorSubcoreMesh(
        core_axis_name="core", subcore_axis_name="subcore")

    @pl.kernel(out_type=jax.ShapeDtypeStruct((n, D), table.dtype), mesh=mesh)
    def gather_kernel(w_hbm, i_hbm, o_hbm):
        def body(i_vmem, o_vmem):
            pltpu.sync_copy(w_hbm.at[i_vmem.at[0]], o_vmem)

        pltpu.emit_pipeline(
            body,
            grid=(n // GATHER_WINDOW,),
            in_specs=[pl.BlockSpec((1, GATHER_WINDOW), lambda i: (0, i))],
            out_specs=[pl.BlockSpec((GATHER_WINDOW, D), lambda i: (i, 0))],
            core_axis_name="subcore",
            dimension_semantics=(pltpu.PARALLEL,),
        )(i_hbm, o_hbm)

    return gather_kernel(table, idx_row)


def _argmin_call(x2_h, w2row, x_h, W):
    nh = x_h.shape[0]
    return pl.pallas_call(
        argmin_block_kernel,
        grid=(nh // ROW_BLK,),
        in_specs=[
            pl.BlockSpec((ROW_BLK, 1), lambda i: (i, 0)),
            pl.BlockSpec((1, K), lambda i: (0, 0)),
            pl.BlockSpec((ROW_BLK, D), lambda i: (i, 0)),
            pl.BlockSpec((K, D), lambda i: (0, 0)),
        ],
        out_specs=pl.BlockSpec((ROW_BLK, 1), lambda i: (i, 0)),
        out_shape=jax.ShapeDtypeStruct((nh, 1), jnp.int32),
    )(x2_h, w2row, x_h, W)


def _residual_call(q_h, x_h):
    nh = x_h.shape[0]
    return pl.pallas_call(
        residual_kernel,
        grid=(4,),
        in_specs=[
            pl.BlockSpec((nh // 4, D), lambda i: (i, 0)),
            pl.BlockSpec((nh // 4, D), lambda i: (i, 0)),
        ],
        out_specs=pl.BlockSpec((nh // 4, D), lambda i: (i, 0)),
        out_shape=jax.ShapeDtypeStruct((nh, D), jnp.float32),
    )(q_h, x_h)


def kernel(x, W):
    flat_x = x.reshape(-1, x.shape[-1])
    n = flat_x.shape[0]
    x2 = jnp.sum(flat_x ** 2, axis=1, keepdims=True)
    w2 = jnp.sum(W ** 2, axis=1)
    w2row = w2.reshape(1, K)

    idx = _argmin_call(x2, w2row, flat_x, W)
    q = sc_gather_rows(W, idx.reshape(1, n))
    out2_flat = _residual_call(q, flat_x)
    return (q.reshape(x.shape), out2_flat.reshape(x.shape))


# speedup vs baseline: 1.2059x; 1.0066x over previous
import jax
import jax.numpy as jnp
from jax.experimental import pallas as pl
from jax.experimental.pallas import tpu as pltpu
from jax.experimental.pallas import tpu_sc as plsc

D = 256
K = 8192
ROW_BLK = 512
GATHER_WINDOW = 128

ARG_CHUNK = 2048


def argmin_block_kernel(x2_ref, w2_ref, x_ref, w_ref, idx_ref):
    x_blk = x_ref[...]
    w = w_ref[...]
    mm2 = jax.lax.dot_general(
        x_blk + x_blk, w, (((1,), (1,)), ((), ())),
        preferred_element_type=jnp.float32)
    scores = (x2_ref[...] + w2_ref[...]) - mm2
    acc_v = None
    for c in range(K // ARG_CHUNK):
        s = scores[:, c * ARG_CHUNK:(c + 1) * ARG_CHUNK]
        m = jnp.min(s, axis=1, keepdims=True)
        i = (jnp.argmin(s, axis=1, keepdims=True).astype(jnp.int32)
             + c * ARG_CHUNK)
        if acc_v is None:
            acc_v = m.astype(jnp.bfloat16).astype(jnp.float32)
            acc_i = i
        else:
            keep = acc_v < m
            tie = acc_v == m
            acc_v = jnp.where(keep, acc_v, m).astype(
                jnp.bfloat16).astype(jnp.float32)
            acc_i = jnp.where(keep | tie, acc_i, i)
    idx_ref[...] = acc_i


def residual_kernel(q_ref, x_ref, o_ref):
    o_ref[...] = (q_ref[...] - x_ref[...]) + x_ref[...]


def sc_gather_rows(table, idx_row):
    n = idx_row.shape[1]
    mesh = plsc.VectorSubcoreMesh(
        core_axis_name="core", subcore_axis_name="subcore")

    @pl.kernel(out_type=jax.ShapeDtypeStruct((n, D), table.dtype), mesh=mesh)
    def gather_kernel(w_hbm, i_hbm, o_hbm):
        def body(i_vmem, o_vmem):
            pltpu.sync_copy(w_hbm.at[i_vmem.at[0]], o_vmem)

        pltpu.emit_pipeline(
            body,
            grid=(n // GATHER_WINDOW,),
            in_specs=[pl.BlockSpec((1, GATHER_WINDOW), lambda i: (0, i))],
            out_specs=[pl.BlockSpec((GATHER_WINDOW, D), lambda i: (i, 0))],
            core_axis_name="subcore",
            dimension_semantics=(pltpu.PARALLEL,),
        )(i_hbm, o_hbm)

    return gather_kernel(table, idx_row)


def _argmin_call(x2_h, w2row, x_h, W):
    nh = x_h.shape[0]
    return pl.pallas_call(
        argmin_block_kernel,
        grid=(nh // ROW_BLK,),
        in_specs=[
            pl.BlockSpec((ROW_BLK, 1), lambda i: (i, 0)),
            pl.BlockSpec((1, K), lambda i: (0, 0)),
            pl.BlockSpec((ROW_BLK, D), lambda i: (i, 0)),
            pl.BlockSpec((K, D), lambda i: (0, 0)),
        ],
        out_specs=pl.BlockSpec((ROW_BLK, 1), lambda i: (i, 0)),
        out_shape=jax.ShapeDtypeStruct((nh, 1), jnp.int32),
    )(x2_h, w2row, x_h, W)


def _residual_call(q_h, x_h):
    nh = x_h.shape[0]
    return pl.pallas_call(
        residual_kernel,
        grid=(4,),
        in_specs=[
            pl.BlockSpec((nh // 4, D), lambda i: (i, 0)),
            pl.BlockSpec((nh // 4, D), lambda i: (i, 0)),
        ],
        out_specs=pl.BlockSpec((nh // 4, D), lambda i: (i, 0)),
        out_shape=jax.ShapeDtypeStruct((nh, D), jnp.float32),
    )(q_h, x_h)


def kernel(x, W):
    flat_x = x.reshape(-1, x.shape[-1])
    n = flat_x.shape[0]
    x2 = jnp.sum(flat_x ** 2, axis=1, keepdims=True)
    w2 = jnp.sum(W ** 2, axis=1)
    w2row = w2.reshape(1, K)

    idx = _argmin_call(x2, w2row, flat_x, W)
    q = sc_gather_rows(W, idx.reshape(1, n))
    out2_flat = _residual_call(q, flat_x)
    return (q.reshape(x.shape), out2_flat.reshape(x.shape))
